# baseline (speedup 1.0000x reference)
import functools

import jax
import jax.numpy as jnp
from jax import lax
from jax.experimental import pallas as pl
from jax.experimental.pallas import tpu as pltpu
from jax.experimental.pallas import tpu_sc as plsc

_N = 10000
_NP = 10240
_E = 320000
_D = 128

_NC = 2
_NS = 16
_NW = _NC * _NS
_EPW = _E // _NW
_K = 128
_CHUNKS = 80
_EPWP = _CHUNKS * _K
_RPT = _NP // _NS


def _segsum_kernel(table, edges, zeros, out, acc, idx_a, idx_b,
                   rows_a, rows_b, sem_ia, sem_ib, sem_ga, sem_gb):
    c = lax.axis_index("c")
    s = lax.axis_index("s")
    pltpu.sync_copy(zeros, acc.at[pl.ds(s * _RPT, _RPT)])

    wid = c * _NS + s
    idx = [idx_a, idx_b]
    rows = [rows_a, rows_b]
    sem_i = [sem_ia, sem_ib]
    sem_g = [sem_ga, sem_gb]
    descs_i = [None, None]
    descs_g = [None, None]

    plsc.subcore_barrier()
    for j in range(_CHUNKS):
        p = j % 2
        pltpu.async_copy(edges.at[wid, j], idx[p], sem_i[p]).wait()
        pltpu.async_copy(table.at[idx[p].at[0]], rows[p], sem_g[p]).wait()
        pltpu.sync_copy(rows[p], acc.at[idx[p].at[1]], add=True)

    plsc.subcore_barrier()
    pltpu.sync_copy(acc.at[pl.ds(s * _RPT, _RPT)],
                    out.at[c, pl.ds(s * _RPT, _RPT)])


_segsum = functools.partial(
    pl.kernel,
    out_type=jax.ShapeDtypeStruct((_NC, _NP, _D), jnp.float32),
    mesh=plsc.VectorSubcoreMesh(core_axis_name="c", subcore_axis_name="s"),
    scratch_types=[
        pltpu.VMEM_SHARED((_NP, _D), jnp.float32),
        pltpu.VMEM((2, _K), jnp.int32),
        pltpu.VMEM((2, _K), jnp.int32),
        pltpu.VMEM((_K, _D), jnp.float32),
        pltpu.VMEM((_K, _D), jnp.float32),
        pltpu.SemaphoreType.DMA,
        pltpu.SemaphoreType.DMA,
        pltpu.SemaphoreType.DMA,
        pltpu.SemaphoreType.DMA,
    ],
)(_segsum_kernel)


_R = 256


def _dense_body(p_ref, x_ref, wr_ref, wt_ref, b_ref, o_ref):
    agg = p_ref[0] + p_ref[1]
    acc = lax.dot_general(agg, wr_ref[...], (((1,), (1,)), ((), ())),
                          preferred_element_type=jnp.float32)
    acc = acc + lax.dot_general(x_ref[...], wt_ref[...],
                                (((1,), (1,)), ((), ())),
                                preferred_element_type=jnp.float32)
    o_ref[...] = jnp.maximum(acc + b_ref[...], 0.0)


def _dense_heads_body(p_ref, x_ref, wr_ref, wt_ref, b_ref, wh_ref, bh_ref,
                      o_ref):
    agg = p_ref[0] + p_ref[1]
    acc = lax.dot_general(agg, wr_ref[...], (((1,), (1,)), ((), ())),
                          preferred_element_type=jnp.float32)
    acc = acc + lax.dot_general(x_ref[...], wt_ref[...],
                                (((1,), (1,)), ((), ())),
                                preferred_element_type=jnp.float32)
    h = jnp.maximum(acc + b_ref[...], 0.0)
    o_ref[...] = lax.dot_general(h, wh_ref[...], (((1,), (1,)), ((), ())),
                                 preferred_element_type=jnp.float32) + bh_ref[...]


def _dense_layer(parts, xp, w_rel, w_root, b):
    return pl.pallas_call(
        _dense_body,
        grid=(_NP // _R,),
        in_specs=[
            pl.BlockSpec((_NC, _R, _D), lambda i: (0, i, 0)),
            pl.BlockSpec((_R, _D), lambda i: (i, 0)),
            pl.BlockSpec((_D, _D), lambda i: (0, 0)),
            pl.BlockSpec((_D, _D), lambda i: (0, 0)),
            pl.BlockSpec((1, _D), lambda i: (0, 0)),
        ],
        out_specs=pl.BlockSpec((_R, _D), lambda i: (i, 0)),
        out_shape=jax.ShapeDtypeStruct((_NP, _D), jnp.float32),
    )(parts, xp, w_rel, w_root, b)


def _dense_layer_heads(parts, xp, w_rel, w_root, b, w_heads, b_heads):
    return pl.pallas_call(
        _dense_heads_body,
        grid=(_NP // _R,),
        in_specs=[
            pl.BlockSpec((_NC, _R, _D), lambda i: (0, i, 0)),
            pl.BlockSpec((_R, _D), lambda i: (i, 0)),
            pl.BlockSpec((_D, _D), lambda i: (0, 0)),
            pl.BlockSpec((_D, _D), lambda i: (0, 0)),
            pl.BlockSpec((1, _D), lambda i: (0, 0)),
            pl.BlockSpec((_D, _D), lambda i: (0, 0)),
            pl.BlockSpec((1, _D), lambda i: (0, 0)),
        ],
        out_specs=pl.BlockSpec((_R, _D), lambda i: (i, 0)),
        out_shape=jax.ShapeDtypeStruct((_NP, _D), jnp.float32),
    )(parts, xp, w_rel, w_root, b, w_heads, b_heads)


def kernel(x, edge_index, W1_rel, b1, W1_root, W2_rel, b2, W2_root,
           Wa, ba, Wo, bo):
    pad_dst = (_N + jnp.arange(_EPWP - _EPW, dtype=jnp.int32)
               % (_NP - _N)).reshape(1, -1).repeat(_NW, axis=0)
    src = edge_index[0].reshape(_NW, _EPW)
    dst = edge_index[1].reshape(_NW, _EPW)
    src = jnp.concatenate(
        [src, jnp.zeros((_NW, _EPWP - _EPW), jnp.int32)], axis=1)
    dst = jnp.concatenate([dst, pad_dst], axis=1)
    edges = jnp.stack([src.reshape(_NW, _CHUNKS, _K),
                       dst.reshape(_NW, _CHUNKS, _K)], axis=2)
    zeros = jnp.zeros((_RPT, _D), jnp.float32)
    xp = jnp.pad(x, ((0, _NP - _N), (0, 0)))

    w_heads = jnp.zeros((_D, _D), jnp.float32)
    w_heads = w_heads.at[:2, :].set(Wa).at[2, :].set(Wo[0])
    b_heads = jnp.zeros((_D,), jnp.float32)
    b_heads = b_heads.at[:2].set(ba).at[2].set(bo[0])

    parts1 = _segsum(x, edges, zeros)
    h1 = _dense_layer(parts1, xp, W1_rel, W1_root, b1.reshape(1, _D))
    parts2 = _segsum(h1, edges, zeros)
    out = _dense_layer_heads(parts2, h1, W2_rel, W2_root, b2.reshape(1, _D),
                             w_heads, b_heads.reshape(1, _D))
    task_allocation = out[:_N, :2]
    task_order = out[:_N, 2:3]
    return (task_allocation, task_order)


# speedup vs baseline: 1.6780x; 1.6780x over previous
import functools

import jax
import jax.numpy as jnp
from jax import lax
from jax.experimental import pallas as pl
from jax.experimental.pallas import tpu as pltpu
from jax.experimental.pallas import tpu_sc as plsc

_N = 10000
_NP = 10240
_E = 320000
_D = 128

_NC = 2
_NS = 16
_NW = _NC * _NS
_EPW = _E // _NW
_K = 80
_CHUNKS = _EPW // _K
_RPT = _NP // _NS


def _segsum_kernel(table, edges, zeros, out, acc, idx_a, idx_b,
                   rows_a, rows_b, sem_ia, sem_ib, sem_ga, sem_gb):
    c = lax.axis_index("c")
    s = lax.axis_index("s")
    pltpu.sync_copy(zeros, acc.at[pl.ds(s * _RPT, _RPT)])

    wid = c * _NS + s
    idx = [idx_a, idx_b]
    rows = [rows_a, rows_b]
    sem_i = [sem_ia, sem_ib]
    sem_g = [sem_ga, sem_gb]
    descs_i = [None, None]
    descs_g = [None, None]

    plsc.subcore_barrier()

    def body(j, carry):
        pltpu.async_copy(edges.at[wid, j], idx_a, sem_ia).wait()
        pltpu.async_copy(table.at[idx_a.at[0]], rows_a, sem_ga).wait()
        pltpu.sync_copy(rows_a, acc.at[idx_a.at[1]], add=True)
        return carry

    lax.fori_loop(0, _CHUNKS, body, 0)

    plsc.subcore_barrier()
    pltpu.sync_copy(acc.at[pl.ds(s * _RPT, _RPT)],
                    out.at[c, pl.ds(s * _RPT, _RPT)])


_segsum = functools.partial(
    pl.kernel,
    out_type=jax.ShapeDtypeStruct((_NC, _NP, _D), jnp.float32),
    mesh=plsc.VectorSubcoreMesh(core_axis_name="c", subcore_axis_name="s"),
    scratch_types=[
        pltpu.VMEM_SHARED((_NP, _D), jnp.float32),
        pltpu.VMEM((2, _K), jnp.int32),
        pltpu.VMEM((2, _K), jnp.int32),
        pltpu.VMEM((_K, _D), jnp.float32),
        pltpu.VMEM((_K, _D), jnp.float32),
        pltpu.SemaphoreType.DMA,
        pltpu.SemaphoreType.DMA,
        pltpu.SemaphoreType.DMA,
        pltpu.SemaphoreType.DMA,
    ],
)(_segsum_kernel)


_R = 256


def _dense_body(p_ref, x_ref, wr_ref, wt_ref, b_ref, o_ref):
    agg = p_ref[0] + p_ref[1]
    acc = lax.dot_general(agg, wr_ref[...], (((1,), (1,)), ((), ())),
                          preferred_element_type=jnp.float32)
    acc = acc + lax.dot_general(x_ref[...], wt_ref[...],
                                (((1,), (1,)), ((), ())),
                                preferred_element_type=jnp.float32)
    o_ref[...] = jnp.maximum(acc + b_ref[...], 0.0)


def _dense_heads_body(p_ref, x_ref, wr_ref, wt_ref, b_ref, wh_ref, bh_ref,
                      o_ref):
    agg = p_ref[0] + p_ref[1]
    acc = lax.dot_general(agg, wr_ref[...], (((1,), (1,)), ((), ())),
                          preferred_element_type=jnp.float32)
    acc = acc + lax.dot_general(x_ref[...], wt_ref[...],
                                (((1,), (1,)), ((), ())),
                                preferred_element_type=jnp.float32)
    h = jnp.maximum(acc + b_ref[...], 0.0)
    o_ref[...] = lax.dot_general(h, wh_ref[...], (((1,), (1,)), ((), ())),
                                 preferred_element_type=jnp.float32) + bh_ref[...]


def _dense_layer(parts, xp, w_rel, w_root, b):
    return pl.pallas_call(
        _dense_body,
        grid=(_NP // _R,),
        in_specs=[
            pl.BlockSpec((_NC, _R, _D), lambda i: (0, i, 0)),
            pl.BlockSpec((_R, _D), lambda i: (i, 0)),
            pl.BlockSpec((_D, _D), lambda i: (0, 0)),
            pl.BlockSpec((_D, _D), lambda i: (0, 0)),
            pl.BlockSpec((1, _D), lambda i: (0, 0)),
        ],
        out_specs=pl.BlockSpec((_R, _D), lambda i: (i, 0)),
        out_shape=jax.ShapeDtypeStruct((_NP, _D), jnp.float32),
    )(parts, xp, w_rel, w_root, b)


def _dense_layer_heads(parts, xp, w_rel, w_root, b, w_heads, b_heads):
    return pl.pallas_call(
        _dense_heads_body,
        grid=(_NP // _R,),
        in_specs=[
            pl.BlockSpec((_NC, _R, _D), lambda i: (0, i, 0)),
            pl.BlockSpec((_R, _D), lambda i: (i, 0)),
            pl.BlockSpec((_D, _D), lambda i: (0, 0)),
            pl.BlockSpec((_D, _D), lambda i: (0, 0)),
            pl.BlockSpec((1, _D), lambda i: (0, 0)),
            pl.BlockSpec((_D, _D), lambda i: (0, 0)),
            pl.BlockSpec((1, _D), lambda i: (0, 0)),
        ],
        out_specs=pl.BlockSpec((_R, _D), lambda i: (i, 0)),
        out_shape=jax.ShapeDtypeStruct((_NP, _D), jnp.float32),
    )(parts, xp, w_rel, w_root, b, w_heads, b_heads)


def kernel(x, edge_index, W1_rel, b1, W1_root, W2_rel, b2, W2_root,
           Wa, ba, Wo, bo):
    src = edge_index[0].reshape(_NW, _CHUNKS, _K)
    dst = edge_index[1].reshape(_NW, _CHUNKS, _K)
    edges = jnp.stack([src, dst], axis=2)
    zeros = jnp.zeros((_RPT, _D), jnp.float32)
    xp = jnp.pad(x, ((0, _NP - _N), (0, 0)))

    w_heads = jnp.zeros((_D, _D), jnp.float32)
    w_heads = w_heads.at[:2, :].set(Wa).at[2, :].set(Wo[0])
    b_heads = jnp.zeros((_D,), jnp.float32)
    b_heads = b_heads.at[:2].set(ba).at[2].set(bo[0])

    parts1 = _segsum(x, edges, zeros)
    h1 = _dense_layer(parts1, xp, W1_rel, W1_root, b1.reshape(1, _D))
    parts2 = _segsum(h1, edges, zeros)
    out = _dense_layer_heads(parts2, h1, W2_rel, W2_root, b2.reshape(1, _D),
                             w_heads, b_heads.reshape(1, _D))
    task_allocation = out[:_N, :2]
    task_order = out[:_N, 2:3]
    return (task_allocation, task_order)


# speedup vs baseline: 2.4586x; 1.4652x over previous
import functools

import jax
import jax.numpy as jnp
from jax import lax
from jax.experimental import pallas as pl
from jax.experimental.pallas import tpu as pltpu
from jax.experimental.pallas import tpu_sc as plsc

_N = 10000
_NP = 10240
_E = 320000
_D = 128

_NC = 2
_NS = 16
_NW = _NC * _NS
_EPW = _E // _NW
_K = 80
_CHUNKS = _EPW // _K
_RPT = _NP // _NS


_G = 4
_GROUPS = _CHUNKS // _G


def _segsum_kernel(table, edges, zeros, out, acc, idxs, rows, sem_i, sem_g):
    c = lax.axis_index("c")
    s = lax.axis_index("s")
    pltpu.sync_copy(zeros, acc.at[pl.ds(s * _RPT, _RPT)])

    wid = c * _NS + s
    plsc.subcore_barrier()

    def body(j, carry):
        jj = j * _G
        di = [pltpu.async_copy(edges.at[wid, jj + g], idxs[g], sem_i[g])
              for g in range(_G)]
        dg = []
        for g in range(_G):
            di[g].wait()
            dg.append(pltpu.async_copy(table.at[idxs[g].at[0]], rows[g],
                                       sem_g[g]))
        for g in range(_G):
            dg[g].wait()
            pltpu.sync_copy(rows[g], acc.at[idxs[g].at[1]], add=True)
        return carry

    lax.fori_loop(0, _GROUPS, body, 0)
    for t in range(_GROUPS * _G, _CHUNKS):
        pltpu.async_copy(edges.at[wid, t], idxs[0], sem_i[0]).wait()
        pltpu.async_copy(table.at[idxs[0].at[0]], rows[0], sem_g[0]).wait()
        pltpu.sync_copy(rows[0], acc.at[idxs[0].at[1]], add=True)

    plsc.subcore_barrier()
    pltpu.sync_copy(acc.at[pl.ds(s * _RPT, _RPT)],
                    out.at[c, pl.ds(s * _RPT, _RPT)])


_segsum = functools.partial(
    pl.kernel,
    out_type=jax.ShapeDtypeStruct((_NC, _NP, _D), jnp.float32),
    mesh=plsc.VectorSubcoreMesh(core_axis_name="c", subcore_axis_name="s"),
    scratch_types=[
        pltpu.VMEM_SHARED((_NP, _D), jnp.float32),
        [pltpu.VMEM((2, _K), jnp.int32) for _ in range(_G)],
        [pltpu.VMEM((_K, _D), jnp.float32) for _ in range(_G)],
        [pltpu.SemaphoreType.DMA for _ in range(_G)],
        [pltpu.SemaphoreType.DMA for _ in range(_G)],
    ],
)(_segsum_kernel)


_R = 256


def _dense_body(p_ref, x_ref, wr_ref, wt_ref, b_ref, o_ref):
    agg = p_ref[0] + p_ref[1]
    acc = lax.dot_general(agg, wr_ref[...], (((1,), (1,)), ((), ())),
                          preferred_element_type=jnp.float32)
    acc = acc + lax.dot_general(x_ref[...], wt_ref[...],
                                (((1,), (1,)), ((), ())),
                                preferred_element_type=jnp.float32)
    o_ref[...] = jnp.maximum(acc + b_ref[...], 0.0)


def _dense_heads_body(p_ref, x_ref, wr_ref, wt_ref, b_ref, wh_ref, bh_ref,
                      o_ref):
    agg = p_ref[0] + p_ref[1]
    acc = lax.dot_general(agg, wr_ref[...], (((1,), (1,)), ((), ())),
                          preferred_element_type=jnp.float32)
    acc = acc + lax.dot_general(x_ref[...], wt_ref[...],
                                (((1,), (1,)), ((), ())),
                                preferred_element_type=jnp.float32)
    h = jnp.maximum(acc + b_ref[...], 0.0)
    o_ref[...] = lax.dot_general(h, wh_ref[...], (((1,), (1,)), ((), ())),
                                 preferred_element_type=jnp.float32) + bh_ref[...]


def _dense_layer(parts, xp, w_rel, w_root, b):
    return pl.pallas_call(
        _dense_body,
        grid=(_NP // _R,),
        in_specs=[
            pl.BlockSpec((_NC, _R, _D), lambda i: (0, i, 0)),
            pl.BlockSpec((_R, _D), lambda i: (i, 0)),
            pl.BlockSpec((_D, _D), lambda i: (0, 0)),
            pl.BlockSpec((_D, _D), lambda i: (0, 0)),
            pl.BlockSpec((1, _D), lambda i: (0, 0)),
        ],
        out_specs=pl.BlockSpec((_R, _D), lambda i: (i, 0)),
        out_shape=jax.ShapeDtypeStruct((_NP, _D), jnp.float32),
    )(parts, xp, w_rel, w_root, b)


def _dense_layer_heads(parts, xp, w_rel, w_root, b, w_heads, b_heads):
    return pl.pallas_call(
        _dense_heads_body,
        grid=(_NP // _R,),
        in_specs=[
            pl.BlockSpec((_NC, _R, _D), lambda i: (0, i, 0)),
            pl.BlockSpec((_R, _D), lambda i: (i, 0)),
            pl.BlockSpec((_D, _D), lambda i: (0, 0)),
            pl.BlockSpec((_D, _D), lambda i: (0, 0)),
            pl.BlockSpec((1, _D), lambda i: (0, 0)),
            pl.BlockSpec((_D, _D), lambda i: (0, 0)),
            pl.BlockSpec((1, _D), lambda i: (0, 0)),
        ],
        out_specs=pl.BlockSpec((_R, _D), lambda i: (i, 0)),
        out_shape=jax.ShapeDtypeStruct((_NP, _D), jnp.float32),
    )(parts, xp, w_rel, w_root, b, w_heads, b_heads)


def kernel(x, edge_index, W1_rel, b1, W1_root, W2_rel, b2, W2_root,
           Wa, ba, Wo, bo):
    src = edge_index[0].reshape(_NW, _CHUNKS, _K)
    dst = edge_index[1].reshape(_NW, _CHUNKS, _K)
    edges = jnp.stack([src, dst], axis=2)
    zeros = jnp.zeros((_RPT, _D), jnp.float32)
    xp = jnp.pad(x, ((0, _NP - _N), (0, 0)))

    w_heads = jnp.zeros((_D, _D), jnp.float32)
    w_heads = w_heads.at[:2, :].set(Wa).at[2, :].set(Wo[0])
    b_heads = jnp.zeros((_D,), jnp.float32)
    b_heads = b_heads.at[:2].set(ba).at[2].set(bo[0])

    parts1 = _segsum(x, edges, zeros)
    h1 = _dense_layer(parts1, xp, W1_rel, W1_root, b1.reshape(1, _D))
    parts2 = _segsum(h1, edges, zeros)
    out = _dense_layer_heads(parts2, h1, W2_rel, W2_root, b2.reshape(1, _D),
                             w_heads, b_heads.reshape(1, _D))
    task_allocation = out[:_N, :2]
    task_order = out[:_N, 2:3]
    return (task_allocation, task_order)


# speedup vs baseline: 2.5956x; 1.0558x over previous
import functools

import jax
import jax.numpy as jnp
from jax import lax
from jax.experimental import pallas as pl
from jax.experimental.pallas import tpu as pltpu
from jax.experimental.pallas import tpu_sc as plsc

_N = 10000
_NP = 10240
_E = 320000
_D = 128

_NC = 2
_NS = 16
_NW = _NC * _NS
_EPW = _E // _NW
_K = 80
_CHUNKS = _EPW // _K
_RPT = _NP // _NS


_G = 4
_GROUPS = _CHUNKS // _G


def _segsum_kernel(table, edges, zeros, out, acc, idxs, rows, sem_i, sem_g):
    c = lax.axis_index("c")
    s = lax.axis_index("s")
    pltpu.sync_copy(zeros, acc.at[pl.ds(s * _RPT, _RPT)])

    wid = c * _NS + s
    plsc.subcore_barrier()

    def body(j, carry):
        jj = j * _G
        di = [pltpu.async_copy(edges.at[wid, jj + g], idxs[g], sem_i[g])
              for g in range(_G)]
        dg = []
        for g in range(_G):
            di[g].wait()
            dg.append(pltpu.async_copy(table.at[idxs[g].at[0]], rows[g],
                                       sem_g[g]))
        for g in range(_G):
            dg[g].wait()
            pltpu.sync_copy(rows[g], acc.at[idxs[g].at[1]], add=True)
        return carry

    lax.fori_loop(0, _GROUPS, body, 0)
    for t in range(_GROUPS * _G, _CHUNKS):
        pltpu.async_copy(edges.at[wid, t], idxs[0], sem_i[0]).wait()
        pltpu.async_copy(table.at[idxs[0].at[0]], rows[0], sem_g[0]).wait()
        pltpu.sync_copy(rows[0], acc.at[idxs[0].at[1]], add=True)

    plsc.subcore_barrier()
    pltpu.sync_copy(acc.at[pl.ds(s * _RPT, _RPT)],
                    out.at[c, pl.ds(s * _RPT, _RPT)])


_segsum = functools.partial(
    pl.kernel,
    out_type=jax.ShapeDtypeStruct((_NC, _NP, _D), jnp.float32),
    mesh=plsc.VectorSubcoreMesh(core_axis_name="c", subcore_axis_name="s"),
    scratch_types=[
        pltpu.VMEM_SHARED((_NP, _D), jnp.float32),
        [pltpu.VMEM((2, _K), jnp.int32) for _ in range(_G)],
        [pltpu.VMEM((_K, _D), jnp.float32) for _ in range(_G)],
        [pltpu.SemaphoreType.DMA for _ in range(_G)],
        [pltpu.SemaphoreType.DMA for _ in range(_G)],
    ],
)(_segsum_kernel)


_R = 400


def _dense_body(p_ref, x_ref, wr_ref, wt_ref, b_ref, o_ref):
    agg = p_ref[0] + p_ref[1]
    acc = lax.dot_general(agg, wr_ref[...], (((1,), (1,)), ((), ())),
                          preferred_element_type=jnp.float32)
    acc = acc + lax.dot_general(x_ref[...], wt_ref[...],
                                (((1,), (1,)), ((), ())),
                                preferred_element_type=jnp.float32)
    o_ref[...] = jnp.maximum(acc + b_ref[...], 0.0)


def _dense_heads_body(p_ref, x_ref, wr_ref, wt_ref, b_ref, wh_ref, bh_ref,
                      o_ref):
    agg = p_ref[0] + p_ref[1]
    acc = lax.dot_general(agg, wr_ref[...], (((1,), (1,)), ((), ())),
                          preferred_element_type=jnp.float32)
    acc = acc + lax.dot_general(x_ref[...], wt_ref[...],
                                (((1,), (1,)), ((), ())),
                                preferred_element_type=jnp.float32)
    h = jnp.maximum(acc + b_ref[...], 0.0)
    o_ref[...] = lax.dot_general(h, wh_ref[...], (((1,), (1,)), ((), ())),
                                 preferred_element_type=jnp.float32) + bh_ref[...]


def _dense_layer(parts, xp, w_rel, w_root, b):
    return pl.pallas_call(
        _dense_body,
        grid=(_N // _R,),
        in_specs=[
            pl.BlockSpec((_NC, _R, _D), lambda i: (0, i, 0)),
            pl.BlockSpec((_R, _D), lambda i: (i, 0)),
            pl.BlockSpec((_D, _D), lambda i: (0, 0)),
            pl.BlockSpec((_D, _D), lambda i: (0, 0)),
            pl.BlockSpec((1, _D), lambda i: (0, 0)),
        ],
        out_specs=pl.BlockSpec((_R, _D), lambda i: (i, 0)),
        out_shape=jax.ShapeDtypeStruct((_N, _D), jnp.float32),
    )(parts, xp, w_rel, w_root, b)


def _dense_layer_heads(parts, xp, w_rel, w_root, b, w_heads, b_heads):
    return pl.pallas_call(
        _dense_heads_body,
        grid=(_N // _R,),
        in_specs=[
            pl.BlockSpec((_NC, _R, _D), lambda i: (0, i, 0)),
            pl.BlockSpec((_R, _D), lambda i: (i, 0)),
            pl.BlockSpec((_D, _D), lambda i: (0, 0)),
            pl.BlockSpec((_D, _D), lambda i: (0, 0)),
            pl.BlockSpec((1, _D), lambda i: (0, 0)),
            pl.BlockSpec((_D, _D), lambda i: (0, 0)),
            pl.BlockSpec((1, _D), lambda i: (0, 0)),
        ],
        out_specs=pl.BlockSpec((_R, _D), lambda i: (i, 0)),
        out_shape=jax.ShapeDtypeStruct((_N, _D), jnp.float32),
    )(parts, xp, w_rel, w_root, b, w_heads, b_heads)


def kernel(x, edge_index, W1_rel, b1, W1_root, W2_rel, b2, W2_root,
           Wa, ba, Wo, bo):
    src = edge_index[0].reshape(_NW, _CHUNKS, _K)
    dst = edge_index[1].reshape(_NW, _CHUNKS, _K)
    edges = jnp.stack([src, dst], axis=2)
    zeros = jnp.zeros((_RPT, _D), jnp.float32)

    w_heads = jnp.zeros((_D, _D), jnp.float32)
    w_heads = w_heads.at[:2, :].set(Wa).at[2, :].set(Wo[0])
    b_heads = jnp.zeros((_D,), jnp.float32)
    b_heads = b_heads.at[:2].set(ba).at[2].set(bo[0])

    parts1 = _segsum(x, edges, zeros)
    h1 = _dense_layer(parts1, x, W1_rel, W1_root, b1.reshape(1, _D))
    parts2 = _segsum(h1, edges, zeros)
    out = _dense_layer_heads(parts2, h1, W2_rel, W2_root, b2.reshape(1, _D),
                             w_heads, b_heads.reshape(1, _D))
    task_allocation = out[:, :2]
    task_order = out[:, 2:3]
    return (task_allocation, task_order)
